# baseline (device time: 31173 ns/iter reference)
import jax
import jax.numpy as jnp
from jax import lax
from jax.experimental import pallas as pl
from jax.experimental.pallas import tpu as pltpu

N_DEV = 8
N_STEPS = 3
ORDERS = ((1, 3, 4), (3, 4, 1))
N_HALVES = 2


def kernel(x, Win0, Wout0, Win1, Wout1, Win2, Wout2):
    b, d = x.shape
    dh = Win0.shape[1]
    rows_out = b // N_DEV
    half = b // N_HALVES
    n_bfly = 2 * N_STEPS * N_HALVES
    n_a2a = N_DEV - 1

    bf16 = jnp.bfloat16
    f32 = jnp.float32

    def body(x_ref, win0_ref, wout0_ref, win1_ref, wout1_ref,
             win2_ref, wout2_ref, out_ref,
             acc_ref, winbf_ref, woutbf_ref,
             sendbuf_ref, comm_ref, stage_ref, a2a_ref,
             send_sems, recv_sems, a2a_send_sems, a2a_recv_sems):
        my = lax.axis_index("i")

        barrier_sem = pltpu.get_barrier_semaphore()
        for m in (1, 3, 4):
            pl.semaphore_signal(
                barrier_sem, inc=1,
                device_id=(my ^ m,),
                device_id_type=pl.DeviceIdType.MESH,
            )
        pl.semaphore_wait(barrier_sem, 3)

        wrefs = [(win0_ref, wout0_ref), (win1_ref, wout1_ref),
                 (win2_ref, wout2_ref)]

        def convert_weights(layer):
            win_ref, wout_ref = wrefs[layer]
            winbf_ref[layer] = win_ref[...].astype(bf16)
            woutbf_ref[layer] = wout_ref[...].astype(bf16)

        def partial_half(layer, xh_bf16):
            hh = jnp.maximum(
                jnp.dot(xh_bf16, winbf_ref[layer],
                        preferred_element_type=f32),
                0.0,
            ).astype(bf16)
            return jnp.dot(hh, woutbf_ref[layer],
                           preferred_element_type=f32)

        def start_bfly(layer, s, hf, val_f32):
            slot = (layer * N_STEPS + s) * N_HALVES + hf
            sendbuf_ref[slot] = val_f32.astype(bf16)
            rdma = pltpu.make_async_remote_copy(
                src_ref=sendbuf_ref.at[slot],
                dst_ref=comm_ref.at[slot],
                send_sem=send_sems.at[slot],
                recv_sem=recv_sems.at[slot],
                device_id=(my ^ ORDERS[hf][s],),
                device_id_type=pl.DeviceIdType.MESH,
            )
            rdma.start()
            return slot, rdma

        convert_weights(0)
        inflight = {}
        for hf in range(N_HALVES):
            p = partial_half(0, x_ref[pl.ds(hf * half, half), :].astype(bf16))
            acc_ref[pl.ds(hf * half, half), :] = p
            inflight[hf] = start_bfly(0, 0, hf, p)
        convert_weights(1)

        for layer in range(2):
            if layer == 1:
                convert_weights(2)
            for s in range(N_STEPS):
                for hf in range(N_HALVES):
                    slot, rdma = inflight[hf]
                    rdma.wait()
                    summ = (acc_ref[pl.ds(hf * half, half), :]
                            + comm_ref[slot].astype(f32))
                    acc_ref[pl.ds(hf * half, half), :] = summ
                    if s < N_STEPS - 1:
                        inflight[hf] = start_bfly(layer, s + 1, hf, summ)
                    else:
                        nxt = partial_half(layer + 1, summ.astype(bf16))
                        acc_ref[pl.ds(hf * half, half), :] = nxt
                        if layer < 1:
                            inflight[hf] = start_bfly(layer + 1, 0, hf, nxt)

        a2a = []
        for o in range(1, N_DEV):
            tgt = my ^ o
            stage_ref[o - 1] = acc_ref[
                pl.ds(tgt * rows_out, rows_out), :
            ].astype(bf16)
            rdma = pltpu.make_async_remote_copy(
                src_ref=stage_ref.at[o - 1],
                dst_ref=a2a_ref.at[o - 1],
                send_sem=a2a_send_sems.at[o - 1],
                recv_sem=a2a_recv_sems.at[o - 1],
                device_id=(tgt,),
                device_id_type=pl.DeviceIdType.MESH,
            )
            rdma.start()
            a2a.append(rdma)
        for rdma in a2a:
            rdma.wait()
        total = acc_ref[pl.ds(my * rows_out, rows_out), :]
        for o in range(1, N_DEV):
            total += a2a_ref[o - 1].astype(f32)
        out_ref[...] = total

    return pl.pallas_call(
        body,
        out_shape=jax.ShapeDtypeStruct((rows_out, d), f32),
        in_specs=[pl.BlockSpec(memory_space=pltpu.VMEM)] * 7,
        out_specs=pl.BlockSpec(memory_space=pltpu.VMEM),
        scratch_shapes=[
            pltpu.VMEM((b, d), f32),
            pltpu.VMEM((3, d, dh), bf16),
            pltpu.VMEM((3, dh, d), bf16),
            pltpu.VMEM((n_bfly, half, d), bf16),
            pltpu.VMEM((n_bfly, half, d), bf16),
            pltpu.VMEM((n_a2a, rows_out, d), bf16),
            pltpu.VMEM((n_a2a, rows_out, d), bf16),
            pltpu.SemaphoreType.DMA((n_bfly,)),
            pltpu.SemaphoreType.DMA((n_bfly,)),
            pltpu.SemaphoreType.DMA((n_a2a,)),
            pltpu.SemaphoreType.DMA((n_a2a,)),
        ],
        compiler_params=pltpu.CompilerParams(collective_id=0),
    )(x, Win0, Wout0, Win1, Wout1, Win2, Wout2)


# device time: 30013 ns/iter; 1.0386x vs baseline; 1.0386x over previous
import jax
import jax.numpy as jnp
from jax import lax
from jax.experimental import pallas as pl
from jax.experimental.pallas import tpu as pltpu

N_DEV = 8
N_STEPS = 3
ORDERS = ((1, 3, 4), (3, 4, 1))
N_HALVES = 2


def kernel(x, Win0, Wout0, Win1, Wout1, Win2, Wout2):
    b, d = x.shape
    dh = Win0.shape[1]
    rows_out = b // N_DEV
    half = b // N_HALVES
    n_bfly = 2 * N_STEPS * N_HALVES
    n_a2a = N_DEV - 1

    bf16 = jnp.bfloat16
    f32 = jnp.float32

    def body(x_ref, win0_ref, wout0_ref, win1_ref, wout1_ref,
             win2_ref, wout2_ref, out_ref,
             acc_ref, sendbuf_ref, comm_ref, stage_ref, a2a_ref,
             send_sems, recv_sems, a2a_send_sems, a2a_recv_sems):
        my = lax.axis_index("i")

        barrier_sem = pltpu.get_barrier_semaphore()
        for m in (1, 3, 4):
            pl.semaphore_signal(
                barrier_sem, inc=1,
                device_id=(my ^ m,),
                device_id_type=pl.DeviceIdType.MESH,
            )
        pl.semaphore_wait(barrier_sem, 3)

        wrefs = [(win0_ref, wout0_ref), (win1_ref, wout1_ref),
                 (win2_ref, wout2_ref)]

        def partial_half(layer, xh_bf16):
            win_ref, wout_ref = wrefs[layer]
            hh = jnp.maximum(
                jnp.dot(xh_bf16, win_ref[...],
                        preferred_element_type=f32),
                0.0,
            ).astype(bf16)
            return jnp.dot(hh, wout_ref[...],
                           preferred_element_type=f32)

        def start_bfly(layer, s, hf, val_f32):
            slot = (layer * N_STEPS + s) * N_HALVES + hf
            sendbuf_ref[slot] = val_f32.astype(bf16)
            rdma = pltpu.make_async_remote_copy(
                src_ref=sendbuf_ref.at[slot],
                dst_ref=comm_ref.at[slot],
                send_sem=send_sems.at[slot],
                recv_sem=recv_sems.at[slot],
                device_id=(my ^ ORDERS[hf][s],),
                device_id_type=pl.DeviceIdType.MESH,
            )
            rdma.start()
            return slot, rdma

        inflight = {}
        for hf in range(N_HALVES):
            p = partial_half(0, x_ref[pl.ds(hf * half, half), :])
            acc_ref[pl.ds(hf * half, half), :] = p
            inflight[hf] = start_bfly(0, 0, hf, p)

        for layer in range(2):
            for s in range(N_STEPS):
                for hf in range(N_HALVES):
                    slot, rdma = inflight[hf]
                    rdma.wait()
                    summ = (acc_ref[pl.ds(hf * half, half), :]
                            + comm_ref[slot].astype(f32))
                    acc_ref[pl.ds(hf * half, half), :] = summ
                    if s < N_STEPS - 1:
                        inflight[hf] = start_bfly(layer, s + 1, hf, summ)
                    else:
                        nxt = partial_half(layer + 1, summ.astype(bf16))
                        acc_ref[pl.ds(hf * half, half), :] = nxt
                        if layer < 1:
                            inflight[hf] = start_bfly(layer + 1, 0, hf, nxt)

        a2a = []
        for o in (6, 2, 5, 7, 1, 3, 4):
            tgt = my ^ o
            stage_ref[o - 1] = acc_ref[
                pl.ds(tgt * rows_out, rows_out), :
            ].astype(bf16)
            rdma = pltpu.make_async_remote_copy(
                src_ref=stage_ref.at[o - 1],
                dst_ref=a2a_ref.at[o - 1],
                send_sem=a2a_send_sems.at[o - 1],
                recv_sem=a2a_recv_sems.at[o - 1],
                device_id=(tgt,),
                device_id_type=pl.DeviceIdType.MESH,
            )
            rdma.start()
            a2a.append(rdma)
        for rdma in a2a:
            rdma.wait()
        total = acc_ref[pl.ds(my * rows_out, rows_out), :]
        for o in range(1, N_DEV):
            total += a2a_ref[o - 1].astype(f32)
        out_ref[...] = total

    return pl.pallas_call(
        body,
        out_shape=jax.ShapeDtypeStruct((rows_out, d), f32),
        in_specs=[pl.BlockSpec(memory_space=pltpu.VMEM)] * 7,
        out_specs=pl.BlockSpec(memory_space=pltpu.VMEM),
        scratch_shapes=[
            pltpu.VMEM((b, d), f32),
            pltpu.VMEM((n_bfly, half, d), bf16),
            pltpu.VMEM((n_bfly, half, d), bf16),
            pltpu.VMEM((n_a2a, rows_out, d), bf16),
            pltpu.VMEM((n_a2a, rows_out, d), bf16),
            pltpu.SemaphoreType.DMA((n_bfly,)),
            pltpu.SemaphoreType.DMA((n_bfly,)),
            pltpu.SemaphoreType.DMA((n_a2a,)),
            pltpu.SemaphoreType.DMA((n_a2a,)),
        ],
        compiler_params=pltpu.CompilerParams(collective_id=0),
    )(
        x.astype(bf16),
        Win0.astype(bf16), Wout0.astype(bf16),
        Win1.astype(bf16), Wout1.astype(bf16),
        Win2.astype(bf16), Wout2.astype(bf16),
    )


# device time: 29458 ns/iter; 1.0582x vs baseline; 1.0188x over previous
import jax
import jax.numpy as jnp
from jax import lax
from jax.experimental import pallas as pl
from jax.experimental.pallas import tpu as pltpu

N_DEV = 8
N_STEPS = 3
ORDERS = ((1, 3, 4), (3, 4, 1))
N_HALVES = 2


def kernel(x, Win0, Wout0, Win1, Wout1, Win2, Wout2):
    b, d = x.shape
    dh = Win0.shape[1]
    rows_out = b // N_DEV
    half = b // N_HALVES
    n_bfly = 2 * N_STEPS * N_HALVES
    n_a2a = N_DEV - 1

    bf16 = jnp.bfloat16
    f32 = jnp.float32

    def body(x_ref, win0_ref, wout0_ref, win1_ref, wout1_ref,
             win2_ref, wout2_ref, out_ref,
             acc_ref, sendbuf_ref, comm_ref, stage_ref, a2a_ref,
             send_sems, recv_sems, a2a_send_sems, a2a_recv_sems):
        my = lax.axis_index("i")

        barrier_sem = pltpu.get_barrier_semaphore()
        for m in (1, 3, 4):
            pl.semaphore_signal(
                barrier_sem, inc=1,
                device_id=(my ^ m,),
                device_id_type=pl.DeviceIdType.MESH,
            )
        pl.semaphore_wait(barrier_sem, 3)

        wrefs = [(win0_ref, wout0_ref), (win1_ref, wout1_ref),
                 (win2_ref, wout2_ref)]

        def partial_half(layer, xh_bf16):
            win_ref, wout_ref = wrefs[layer]
            hh = jnp.maximum(
                jnp.dot(xh_bf16, win_ref[...],
                        preferred_element_type=f32),
                0.0,
            ).astype(bf16)
            return jnp.dot(hh, wout_ref[...],
                           preferred_element_type=f32)

        def start_bfly(layer, s, hf, val_f32):
            slot = (layer * N_STEPS + s) * N_HALVES + hf
            sendbuf_ref[slot] = val_f32.astype(bf16)
            rdma = pltpu.make_async_remote_copy(
                src_ref=sendbuf_ref.at[slot],
                dst_ref=comm_ref.at[slot],
                send_sem=send_sems.at[slot],
                recv_sem=recv_sems.at[slot],
                device_id=(my ^ ORDERS[hf][s],),
                device_id_type=pl.DeviceIdType.MESH,
            )
            rdma.start()
            return slot, rdma

        inflight = {}
        for hf in range(N_HALVES):
            p = partial_half(0, x_ref[pl.ds(hf * half, half), :])
            acc_ref[pl.ds(hf * half, half), :] = p
            inflight[hf] = start_bfly(0, 0, hf, p)

        for layer in range(2):
            for s in range(N_STEPS):
                for hf in range(N_HALVES):
                    slot, rdma = inflight[hf]
                    rdma.wait()
                    summ = (acc_ref[pl.ds(hf * half, half), :]
                            + comm_ref[slot].astype(f32))
                    acc_ref[pl.ds(hf * half, half), :] = summ
                    if s < N_STEPS - 1:
                        inflight[hf] = start_bfly(layer, s + 1, hf, summ)
                    else:
                        nxt = partial_half(layer + 1, summ.astype(bf16))
                        acc_ref[pl.ds(hf * half, half), :] = nxt
                        if layer < 1:
                            inflight[hf] = start_bfly(layer + 1, 0, hf, nxt)
                        else:
                            for j in range(N_DEV // N_HALVES):
                                q = hf * (N_DEV // N_HALVES) + j
                                o = my ^ q
                                blk = nxt[
                                    j * rows_out:(j + 1) * rows_out, :
                                ].astype(bf16)

                                @pl.when(q != my)
                                def _(q=q, o=o, blk=blk):
                                    stage_ref[o - 1] = blk
                                    pltpu.make_async_remote_copy(
                                        src_ref=stage_ref.at[o - 1],
                                        dst_ref=a2a_ref.at[o - 1],
                                        send_sem=a2a_send_sems.at[o - 1],
                                        recv_sem=a2a_recv_sems.at[o - 1],
                                        device_id=(q,),
                                        device_id_type=pl.DeviceIdType.MESH,
                                    ).start()

        a2a = []
        for o in range(1, N_DEV):
            a2a.append(pltpu.make_async_remote_copy(
                src_ref=stage_ref.at[o - 1],
                dst_ref=a2a_ref.at[o - 1],
                send_sem=a2a_send_sems.at[o - 1],
                recv_sem=a2a_recv_sems.at[o - 1],
                device_id=(my ^ o,),
                device_id_type=pl.DeviceIdType.MESH,
            ))
        for rdma in a2a:
            rdma.wait_recv()
        total = acc_ref[pl.ds(my * rows_out, rows_out), :]
        for o in range(1, N_DEV):
            total += a2a_ref[o - 1].astype(f32)
        out_ref[...] = total
        for rdma in a2a:
            rdma.wait_send()

    return pl.pallas_call(
        body,
        out_shape=jax.ShapeDtypeStruct((rows_out, d), f32),
        in_specs=[pl.BlockSpec(memory_space=pltpu.VMEM)] * 7,
        out_specs=pl.BlockSpec(memory_space=pltpu.VMEM),
        scratch_shapes=[
            pltpu.VMEM((b, d), f32),
            pltpu.VMEM((n_bfly, half, d), bf16),
            pltpu.VMEM((n_bfly, half, d), bf16),
            pltpu.VMEM((n_a2a, rows_out, d), bf16),
            pltpu.VMEM((n_a2a, rows_out, d), bf16),
            pltpu.SemaphoreType.DMA((n_bfly,)),
            pltpu.SemaphoreType.DMA((n_bfly,)),
            pltpu.SemaphoreType.DMA((n_a2a,)),
            pltpu.SemaphoreType.DMA((n_a2a,)),
        ],
        compiler_params=pltpu.CompilerParams(collective_id=0),
    )(
        x.astype(bf16),
        Win0.astype(bf16), Wout0.astype(bf16),
        Win1.astype(bf16), Wout1.astype(bf16),
        Win2.astype(bf16), Wout2.astype(bf16),
    )
